# Initial kernel scaffold; baseline (speedup 1.0000x reference)
import jax
import jax.numpy as jnp
from jax.experimental import pallas as pl


def kernel(feat, edge_index, etypes, weight):
    raise NotImplementedError("write your pallas kernel here")


# speedup vs baseline: 4.5586x; 4.5586x over previous
import functools

import jax
import jax.numpy as jnp
from jax import lax
from jax.experimental import pallas as pl
from jax.experimental.pallas import tpu as pltpu
from jax.experimental.pallas import tpu_sc as plsc

N = 10000
E = 160000
D = 256
R = 8
H = 128
NC = 2
NT = 16
CH = 80
EPT = E // NT
NCH = EPT // CH
BN = 1000
NB = N // BN
ZR = 200
NZ = N // ZR


def _mm_body(feat_ref, w_ref, t_ref):
    t_ref[...] = jnp.dot(feat_ref[...], w_ref[0],
                         preferred_element_type=jnp.float32)


def _transform(feat, weight):
    return pl.pallas_call(
        _mm_body,
        grid=(NB, NC, R),
        in_specs=[
            pl.BlockSpec((BN, D), lambda i, c, r: (i, 0)),
            pl.BlockSpec((1, D, H), lambda i, c, r: (r, 0, c)),
        ],
        out_specs=pl.BlockSpec((BN, H), lambda i, c, r: (c * R * NB + r * NB + i, 0)),
        out_shape=jax.ShapeDtypeStruct((NC * R * N, H), jnp.float32),
    )(feat, weight)


def _sc_body(t_hbm, src_hbm, et_hbm, dst_hbm, out_hbm,
             accum, src_v, et_v, dst_v, key_v, rows_v, stage_v, sem):
    c = lax.axis_index("c")
    s = lax.axis_index("s")

    def _zrow(j, carry):
        for k in range(H // 16):
            stage_v[j, pl.ds(k * 16, 16)] = jnp.zeros((16,), jnp.float32)
        return carry
    lax.fori_loop(0, ZR, _zrow, 0)
    nch = (NZ - s + NT - 1) // NT

    def _zchunk(i, carry):
        pltpu.sync_copy(stage_v, accum.at[pl.ds((s + i * NT) * ZR, ZR)])
        return carry
    lax.fori_loop(0, nch, _zchunk, 0)
    plsc.subcore_barrier()

    ebase = s * EPT
    c_off = c * (R * N)

    def _chunk(i, carry):
        e0 = ebase + i * CH
        pltpu.sync_copy(src_hbm.at[pl.ds(e0, CH)], src_v)
        pltpu.sync_copy(et_hbm.at[pl.ds(e0, CH)], et_v)
        pltpu.sync_copy(dst_hbm.at[pl.ds(e0, CH)], dst_v)
        for k in range(CH // 16):
            sl = pl.ds(k * 16, 16)
            key_v[sl] = et_v[sl] * N + src_v[sl] + c_off
        pltpu.async_copy(t_hbm.at[key_v], rows_v, sem).wait()
        pltpu.sync_copy(rows_v, accum.at[dst_v], add=True)
        return carry

    lax.fori_loop(0, NCH, _chunk, 0)
    plsc.subcore_barrier()

    def _drain(i, carry):
        r0 = (s + i * NT) * ZR
        pltpu.sync_copy(accum.at[pl.ds(r0, ZR)], stage_v)
        pltpu.sync_copy(stage_v, out_hbm.at[c].at[pl.ds(r0, ZR)])
        return carry
    lax.fori_loop(0, nch, _drain, 0)


def _aggregate(t, src, et, dst):
    mesh = plsc.VectorSubcoreMesh(core_axis_name="c", subcore_axis_name="s")
    f = pl.kernel(
        _sc_body,
        mesh=mesh,
        out_type=jax.ShapeDtypeStruct((NC, N, H), jnp.float32),
        scratch_types=[
            pltpu.VMEM_SHARED((N, H), jnp.float32),
            pltpu.VMEM((CH,), jnp.int32),
            pltpu.VMEM((CH,), jnp.int32),
            pltpu.VMEM((CH,), jnp.int32),
            pltpu.VMEM((CH,), jnp.int32),
            pltpu.VMEM((CH, H), jnp.float32),
            pltpu.VMEM((ZR, H), jnp.float32),
            pltpu.SemaphoreType.DMA,
        ],
    )
    return f(t, src, et, dst)


def kernel(feat, edge_index, etypes, weight):
    t = _transform(feat, weight)
    out2 = _aggregate(t, edge_index[0], etypes, edge_index[1])
    return out2.transpose(1, 0, 2).reshape(N, D)


# speedup vs baseline: 7.4844x; 1.6418x over previous
import functools

import jax
import jax.numpy as jnp
from jax import lax
from jax.experimental import pallas as pl
from jax.experimental.pallas import tpu as pltpu
from jax.experimental.pallas import tpu_sc as plsc

N = 10000
E = 160000
D = 256
R = 8
H = 128
NC = 2
NT = 16
CH = 80
EPT = E // NT
NCH = EPT // CH
BN = 1000
NB = N // BN
ZR = 200
NZ = N // ZR


def _mm_body(feat_ref, w_ref, t_ref):
    t_ref[...] = jnp.dot(feat_ref[...], w_ref[0],
                         preferred_element_type=jnp.float32)


def _transform(feat, weight):
    return pl.pallas_call(
        _mm_body,
        grid=(NB, NC, R),
        in_specs=[
            pl.BlockSpec((BN, D), lambda i, c, r: (i, 0)),
            pl.BlockSpec((1, D, H), lambda i, c, r: (r, 0, c)),
        ],
        out_specs=pl.BlockSpec((BN, H), lambda i, c, r: (c * R * NB + r * NB + i, 0)),
        out_shape=jax.ShapeDtypeStruct((NC * R * N, H), jnp.float32),
    )(feat, weight)


def _sc_body(t_hbm, src_hbm, et_hbm, dst_hbm, out_hbm,
             accum, src_v, et_v, dst_v, key_v, sdst_v, rows_v, stage_v,
             sem_i0, sem_i1, sem_g0, sem_g1):
    c = lax.axis_index("c")
    s = lax.axis_index("s")
    sem_i = (sem_i0, sem_i1)
    sem_g = (sem_g0, sem_g1)

    def _zrow(j, carry):
        for k in range(H // 16):
            stage_v[j, pl.ds(k * 16, 16)] = jnp.zeros((16,), jnp.float32)
        return carry
    lax.fori_loop(0, ZR, _zrow, 0)
    nch = (NZ - s + NT - 1) // NT

    def _zchunk(i, carry):
        pltpu.sync_copy(stage_v, accum.at[pl.ds((s + i * NT) * ZR, ZR)])
        return carry
    lax.fori_loop(0, nch, _zchunk, 0)
    plsc.subcore_barrier()

    ebase = s * EPT
    c_off = c * (R * N)

    def _idx_start(g, b):
        e0 = ebase + g * CH
        pltpu.make_async_copy(src_hbm.at[pl.ds(e0, CH)], src_v.at[b], sem_i[b]).start()
        pltpu.make_async_copy(et_hbm.at[pl.ds(e0, CH)], et_v.at[b], sem_i[b]).start()
        pltpu.make_async_copy(dst_hbm.at[pl.ds(e0, CH)], dst_v.at[b], sem_i[b]).start()

    def _idx_wait(g, b):
        e0 = ebase + g * CH
        pltpu.make_async_copy(src_hbm.at[pl.ds(e0, CH)], src_v.at[b], sem_i[b]).wait()
        pltpu.make_async_copy(et_hbm.at[pl.ds(e0, CH)], et_v.at[b], sem_i[b]).wait()
        pltpu.make_async_copy(dst_hbm.at[pl.ds(e0, CH)], dst_v.at[b], sem_i[b]).wait()

    def _keys(b):
        for k in range(CH // 16):
            sl = pl.ds(k * 16, 16)
            key_v[b, sl] = et_v[b, sl] * N + src_v[b, sl] + c_off
            sdst_v[b, sl] = dst_v[b, sl]

    def _gather_start(b):
        pltpu.make_async_copy(t_hbm.at[key_v.at[b]], rows_v.at[b], sem_g[b]).start()

    def _gather_wait(b):
        pltpu.make_async_copy(t_hbm.at[key_v.at[b]], rows_v.at[b], sem_g[b]).wait()

    def _scatter(b):
        pltpu.sync_copy(rows_v.at[b], accum.at[sdst_v.at[b]], add=True)

    _idx_start(0, 0)
    _idx_start(1, 1)
    _idx_wait(0, 0)
    _keys(0)
    _gather_start(0)
    _idx_start(2, 0)

    def _pair(p, carry):
        ga = 1 + 2 * p
        for b, g in ((1, ga), (0, ga + 1)):
            _idx_wait(g, b)
            _keys(b)
            _gather_wait(1 - b)
            _gather_start(b)

            @pl.when(g + 2 < NCH)
            def _():
                _idx_start(g + 2, b)
            _scatter(1 - b)
        return carry

    lax.fori_loop(0, (NCH - 1) // 2, _pair, 0)
    _gather_wait(0)
    _scatter(0)
    plsc.subcore_barrier()

    def _drain(i, carry):
        r0 = (s + i * NT) * ZR
        pltpu.sync_copy(accum.at[pl.ds(r0, ZR)], stage_v)
        pltpu.sync_copy(stage_v, out_hbm.at[c].at[pl.ds(r0, ZR)])
        return carry
    lax.fori_loop(0, nch, _drain, 0)


def _aggregate(t, src, et, dst):
    mesh = plsc.VectorSubcoreMesh(core_axis_name="c", subcore_axis_name="s")
    f = pl.kernel(
        _sc_body,
        mesh=mesh,
        out_type=jax.ShapeDtypeStruct((NC, N, H), jnp.float32),
        scratch_types=[
            pltpu.VMEM_SHARED((N, H), jnp.float32),
            pltpu.VMEM((2, CH), jnp.int32),
            pltpu.VMEM((2, CH), jnp.int32),
            pltpu.VMEM((2, CH), jnp.int32),
            pltpu.VMEM((2, CH), jnp.int32),
            pltpu.VMEM((2, CH), jnp.int32),
            pltpu.VMEM((2, CH, H), jnp.float32),
            pltpu.VMEM((ZR, H), jnp.float32),
            pltpu.SemaphoreType.DMA,
            pltpu.SemaphoreType.DMA,
            pltpu.SemaphoreType.DMA,
            pltpu.SemaphoreType.DMA,
        ],
    )
    return f(t, src, et, dst)


def kernel(feat, edge_index, etypes, weight):
    t = _transform(feat, weight)
    out2 = _aggregate(t, edge_index[0], etypes, edge_index[1])
    return out2.transpose(1, 0, 2).reshape(N, D)


# speedup vs baseline: 9.3690x; 1.2518x over previous
import functools

import jax
import jax.numpy as jnp
from jax import lax
from jax.experimental import pallas as pl
from jax.experimental.pallas import tpu as pltpu
from jax.experimental.pallas import tpu_sc as plsc

N = 10000
E = 160000
D = 256
R = 8
H = 128
NC = 2
NT = 16
CH = 80
EPT = E // NT
NCH = EPT // CH
NBUF = 4
BN = 1000
NB = N // BN
ZR = 80
NZ = N // ZR


def _mm_body(feat_ref, w_ref, t_ref):
    t_ref[...] = jnp.dot(feat_ref[...], w_ref[0],
                         preferred_element_type=jnp.float32)


def _transform(feat, weight):
    return pl.pallas_call(
        _mm_body,
        grid=(NB, NC, R),
        in_specs=[
            pl.BlockSpec((BN, D), lambda i, c, r: (i, 0)),
            pl.BlockSpec((1, D, H), lambda i, c, r: (r, 0, c)),
        ],
        out_specs=pl.BlockSpec((BN, H), lambda i, c, r: (c * R * NB + r * NB + i, 0)),
        out_shape=jax.ShapeDtypeStruct((NC * R * N, H), jnp.float32),
    )(feat, weight)


def _sc_body(t_hbm, src_hbm, et_hbm, dst_hbm, out_hbm,
             accum, src_v, et_v, dst_v, key_v, sdst_v, rows_v,
             *sems):
    c = lax.axis_index("c")
    s = lax.axis_index("s")
    sem_i = sems[0:NBUF]
    sem_g = sems[NBUF:2 * NBUF]
    sem_s = sems[2 * NBUF:3 * NBUF]

    def _zrow(j, carry):
        for k in range(H // 16):
            rows_v[0, j, pl.ds(k * 16, 16)] = jnp.zeros((16,), jnp.float32)
        return carry
    lax.fori_loop(0, ZR, _zrow, 0)
    nzc = (NZ - s + NT - 1) // NT

    def _zchunk(i, carry):
        pltpu.sync_copy(rows_v.at[0], accum.at[pl.ds((s + i * NT) * ZR, ZR)])
        return carry
    lax.fori_loop(0, nzc, _zchunk, 0)
    plsc.subcore_barrier()

    ebase = s * EPT
    c_off = c * (R * N)

    def _idx_start(g, b):
        e0 = ebase + g * CH
        pltpu.make_async_copy(src_hbm.at[pl.ds(e0, CH)], src_v.at[b], sem_i[b]).start()
        pltpu.make_async_copy(et_hbm.at[pl.ds(e0, CH)], et_v.at[b], sem_i[b]).start()
        pltpu.make_async_copy(dst_hbm.at[pl.ds(e0, CH)], dst_v.at[b], sem_i[b]).start()

    def _idx_wait(g, b):
        e0 = ebase + g * CH
        pltpu.make_async_copy(src_hbm.at[pl.ds(e0, CH)], src_v.at[b], sem_i[b]).wait()
        pltpu.make_async_copy(et_hbm.at[pl.ds(e0, CH)], et_v.at[b], sem_i[b]).wait()
        pltpu.make_async_copy(dst_hbm.at[pl.ds(e0, CH)], dst_v.at[b], sem_i[b]).wait()

    def _keys(b):
        for k in range(CH // 16):
            sl = pl.ds(k * 16, 16)
            key_v[b, sl] = et_v[b, sl] * N + src_v[b, sl] + c_off
            sdst_v[b, sl] = dst_v[b, sl]

    def _gather_start(b):
        pltpu.make_async_copy(t_hbm.at[key_v.at[b]], rows_v.at[b], sem_g[b]).start()

    def _gather_wait(b):
        pltpu.make_async_copy(t_hbm.at[key_v.at[b]], rows_v.at[b], sem_g[b]).wait()

    def _scat_start(b):
        pltpu.make_async_copy(rows_v.at[b], accum.at[sdst_v.at[b]],
                              sem_s[b]).start(add=True)

    def _scat_wait(b):
        pltpu.make_async_copy(rows_v.at[b], accum.at[sdst_v.at[b]],
                              sem_s[b]).wait()

    for b in range(NBUF):
        _idx_start(b, b)
    for g in range(2):
        _idx_wait(g, g)
        _keys(g)
        _gather_start(g)
        _idx_start(g + NBUF, g)
    for g in (2, 3):
        _gather_wait(g - 2)
        _scat_start(g - 2)
        _idx_wait(g, g)
        _keys(g)
        _gather_start(g)
        _idx_start(g + NBUF, g)

    def _quad(p, carry):
        g0 = NBUF + NBUF * p
        for j in range(NBUF):
            g = g0 + j
            _gather_wait((j + 2) % NBUF)
            _scat_start((j + 2) % NBUF)
            _idx_wait(g, j)
            _scat_wait(j)
            _keys(j)
            _gather_start(j)

            @pl.when(g + NBUF < NCH)
            def _():
                _idx_start(g + NBUF, j)
        return carry

    lax.fori_loop(0, (NCH - NBUF - 1) // NBUF, _quad, 0)

    _gather_wait(2)
    _scat_start(2)
    _idx_wait(NCH - 1, 0)
    _scat_wait(0)
    _keys(0)
    _gather_start(0)
    _gather_wait(3)
    _scat_start(3)
    _gather_wait(0)
    _scat_start(0)
    for b in (1, 2, 3, 0):
        _scat_wait(b)
    plsc.subcore_barrier()

    def _drain(i, carry):
        r0 = (s + i * NT) * ZR
        pltpu.sync_copy(accum.at[pl.ds(r0, ZR)], rows_v.at[0])
        pltpu.sync_copy(rows_v.at[0], out_hbm.at[pl.ds(r0, ZR), pl.ds(c * H, H)])
        return carry
    lax.fori_loop(0, nzc, _drain, 0)


def _aggregate(t, src, et, dst):
    mesh = plsc.VectorSubcoreMesh(core_axis_name="c", subcore_axis_name="s")
    f = pl.kernel(
        _sc_body,
        mesh=mesh,
        out_type=jax.ShapeDtypeStruct((N, D), jnp.float32),
        scratch_types=[
            pltpu.VMEM_SHARED((N, H), jnp.float32),
            pltpu.VMEM((NBUF, CH), jnp.int32),
            pltpu.VMEM((NBUF, CH), jnp.int32),
            pltpu.VMEM((NBUF, CH), jnp.int32),
            pltpu.VMEM((NBUF, CH), jnp.int32),
            pltpu.VMEM((NBUF, CH), jnp.int32),
            pltpu.VMEM((NBUF, CH, H), jnp.float32),
        ] + [pltpu.SemaphoreType.DMA] * (3 * NBUF),
    )
    return f(t, src, et, dst)


def kernel(feat, edge_index, etypes, weight):
    t = _transform(feat, weight)
    return _aggregate(t, edge_index[0], etypes, edge_index[1])
